# Initial kernel scaffold; baseline (speedup 1.0000x reference)
import jax
import jax.numpy as jnp
from jax.experimental import pallas as pl


def kernel(xyz, points, new_xyz, W0, b0, g0, be0, W1, b1, g1, be1, W2, b2, g2, be2):
    raise NotImplementedError("write your pallas kernel here")


# speedup vs baseline: 8.1830x; 8.1830x over previous
import functools

import jax
import jax.numpy as jnp
from jax import lax
from jax.experimental import pallas as pl
from jax.experimental.pallas import tpu as pltpu
from jax.experimental.pallas import tpu_sc as plsc

_R2 = 0.2 * 0.2
_K = 32
_EPS = 1e-5
_L = 16


def _d2_body(q_ref, p_ref, o_ref):
    q = q_ref[0]
    p = p_ref[0]
    dt = jax.lax.dot_general(q, p, (((1,), (1,)), ((), ())),
                             preferred_element_type=jnp.float32)
    qq = q * q
    pp = p * p
    qn = (qq[:, 0:1] + qq[:, 1:2]) + qq[:, 2:3]
    pn = (pp[:, 0:1] + pp[:, 1:2]) + pp[:, 2:3]
    o_ref[0] = (qn + pn.T) - 2.0 * dt


def _d2_call(new_xyz, xyz):
    B, N, _ = xyz.shape
    S = new_xyz.shape[1]
    return pl.pallas_call(
        _d2_body,
        grid=(B,),
        in_specs=[pl.BlockSpec((1, S, 3), lambda i: (i, 0, 0)),
                  pl.BlockSpec((1, N, 3), lambda i: (i, 0, 0))],
        out_specs=pl.BlockSpec((1, S, N), lambda i: (i, 0, 0)),
        out_shape=jax.ShapeDtypeStruct((B, S, N), jnp.float32),
    )(new_xyz, xyz)


def _sc_ball_gather(d2_flat, xyz_t, nxyz_t, pts_flat, B, N, S, Cin):
    QW = (B * S) // 32
    NCHUNK = N // _L
    R = B * S * _K

    mesh = plsc.VectorSubcoreMesh(core_axis_name="c", subcore_axis_name="s")

    def body(d2_hbm, xyz_hbm, nxyz_hbm, pts_hbm, feat_out, rel_out,
             xyz_v, nx_v, buf_v, idx_v, rel_v, feat_v, idxg_v, d2_v, sem):
        wid = lax.axis_index("s") * 2 + lax.axis_index("c")
        b = wid // 2
        half = wid % 2
        qbase = b * S + half * QW

        pltpu.sync_copy(xyz_hbm.at[b], xyz_v)
        for c in range(3):
            pltpu.sync_copy(nxyz_hbm.at[b, pl.ds(c * S + half * QW, QW)],
                            nx_v.at[pl.ds(c * QW, QW)])

        zf = jnp.zeros((_L,), jnp.float32)
        for i in range(_K * 8 // _L):
            rel_v[pl.ds(i * _L, _L)] = zf

        lane = lax.iota(jnp.int32, _L)
        zero16 = jnp.zeros((_L,), jnp.int32)

        def per_query(q, _):
            qsp = lax.broadcast(q, (_L,))
            qx = plsc.load_gather(nx_v, [qsp])
            qy = plsc.load_gather(nx_v, [qsp + QW])
            qz = plsc.load_gather(nx_v, [qsp + 2 * QW])
            pltpu.sync_copy(d2_hbm.at[qbase + q], d2_v)

            def scan_chunk(i, cur):
                d2 = d2_v[pl.ds(i * _L, _L)]
                m = d2 <= _R2
                iv = lane + i * _L
                off = jnp.minimum(cur, _K + _L)
                ms = jnp.logical_and(m, lax.broadcast(cur <= _K + _L, (_L,)))
                plsc.store_compressed(buf_v.at[pl.ds(off, _L)], iv, mask=ms)
                cnt = jnp.sum(m.astype(jnp.int32))
                return cur + cnt

            cursor = lax.fori_loop(0, NCHUNK, scan_chunk, 0)

            cvec = lax.broadcast(cursor, (_L,))
            far = zero16 + (N - 1)

            for h in range(_K // _L):
                jvec = lane + h * _L
                jsel = jnp.where(jvec < cvec, jvec, 0)
                gihalf = plsc.load_gather(buf_v, [jsel])
                gi = jnp.where(cvec > 0, gihalf, far)
                gx = plsc.load_gather(xyz_v, [gi])
                gy = plsc.load_gather(xyz_v, [gi + N])
                gz = plsc.load_gather(xyz_v, [gi + 2 * N])
                pos = (jvec * 8)
                plsc.store_scatter(rel_v, [pos], gx - qx)
                plsc.store_scatter(rel_v, [pos + 1], gy - qy)
                plsc.store_scatter(rel_v, [pos + 2], gz - qz)
                idx_v[pl.ds(q * _K + h * _L, _L)] = gi + b * N

            pltpu.sync_copy(rel_v, rel_out.at[pl.ds((qbase + q) * _K * 8, _K * 8)])
            return _

        lax.fori_loop(0, QW, per_query, 0)

        GR = 128
        rowbase = qbase * _K

        def per_gather(g, _):
            for t in range(GR // _L):
                idxg_v[pl.ds(t * _L, _L)] = idx_v[pl.ds(g * GR + t * _L, _L)]
            pltpu.async_copy(pts_hbm.at[idxg_v], feat_v, sem).wait()
            pltpu.sync_copy(feat_v, feat_out.at[pl.ds(rowbase + g * GR, GR)])
            return _

        lax.fori_loop(0, (QW * _K) // GR, per_gather, 0)

    call = pl.kernel(
        body,
        out_type=(
            jax.ShapeDtypeStruct((R, Cin), jnp.float32),
            jax.ShapeDtypeStruct((R * 8,), jnp.float32),
        ),
        mesh=mesh,
        compiler_params=pltpu.CompilerParams(needs_layout_passes=False,
                                             use_tc_tiling_on_sc=False),
        scratch_types=[
            pltpu.VMEM((3 * N,), jnp.float32),
            pltpu.VMEM((3 * QW,), jnp.float32),
            pltpu.VMEM((64,), jnp.int32),
            pltpu.VMEM((QW * _K,), jnp.int32),
            pltpu.VMEM((_K * 8,), jnp.float32),
            pltpu.VMEM((128, Cin), jnp.float32),
            pltpu.VMEM((128,), jnp.int32),
            pltpu.VMEM((N,), jnp.float32),
            pltpu.SemaphoreType.DMA,
        ],
    )
    return call(d2_flat, xyz_t, nxyz_t, pts_flat)


_BLK = 8192


def _h_first(feat, rel, w0f, w0r, b0):
    h = jnp.dot(feat, w0f, preferred_element_type=jnp.float32)
    h = h + jnp.dot(rel, w0r, preferred_element_type=jnp.float32)
    return h + b0


def _stats_tail(i, h, out_ref, acc_ref):
    @pl.when(i == 0)
    def _():
        acc_ref[...] = jnp.zeros_like(acc_ref)

    acc_ref[0:1] += jnp.sum(h, axis=0, keepdims=True)
    acc_ref[1:2] += jnp.sum(h * h, axis=0, keepdims=True)

    @pl.when(i == pl.num_programs(0) - 1)
    def _():
        out_ref[...] = acc_ref[...]


def _pass_a_body(feat_ref, rel_ref, w0f, w0r, b0, out_ref, acc_ref):
    i = pl.program_id(0)
    h = _h_first(feat_ref[...], rel_ref[...], w0f[...], w0r[...], b0[...])
    _stats_tail(i, h, out_ref, acc_ref)


def _pass_b_body(feat_ref, rel_ref, w0f, w0r, b0, sc1, sh1, w1, b1,
                 out_ref, acc_ref):
    i = pl.program_id(0)
    h = _h_first(feat_ref[...], rel_ref[...], w0f[...], w0r[...], b0[...])
    y1 = jnp.maximum(h * sc1[...] + sh1[...], 0.0)
    h2 = jnp.dot(y1, w1[...], preferred_element_type=jnp.float32) + b1[...]
    _stats_tail(i, h2, out_ref, acc_ref)


def _pass_c_body(feat_ref, rel_ref, w0f, w0r, b0, sc1, sh1, w1, b1, sc2, sh2,
                 w2, b2, out_ref, acc_ref):
    i = pl.program_id(0)
    h = _h_first(feat_ref[...], rel_ref[...], w0f[...], w0r[...], b0[...])
    y1 = jnp.maximum(h * sc1[...] + sh1[...], 0.0)
    h2 = jnp.dot(y1, w1[...], preferred_element_type=jnp.float32) + b1[...]
    y2 = jnp.maximum(h2 * sc2[...] + sh2[...], 0.0)
    h3 = jnp.dot(y2, w2[...], preferred_element_type=jnp.float32) + b2[...]
    _stats_tail(i, h3, out_ref, acc_ref)


def _pass_d_body(feat_ref, rel_ref, w0f, w0r, b0, sc1, sh1, w1, b1, sc2, sh2,
                 w2, b2, sc3, sh3, out_ref):
    h = _h_first(feat_ref[...], rel_ref[...], w0f[...], w0r[...], b0[...])
    y1 = jnp.maximum(h * sc1[...] + sh1[...], 0.0)
    h2 = jnp.dot(y1, w1[...], preferred_element_type=jnp.float32) + b1[...]
    y2 = jnp.maximum(h2 * sc2[...] + sh2[...], 0.0)
    h3 = jnp.dot(y2, w2[...], preferred_element_type=jnp.float32) + b2[...]
    y3 = jnp.maximum(h3 * sc3[...] + sh3[...], 0.0)
    co = y3.shape[-1]
    out_ref[...] = jnp.max(y3.reshape(_BLK // _K, _K, co), axis=1)


def _full(shape):
    return pl.BlockSpec(shape, lambda i: (0,) * len(shape))


def _stats_call(body, n_extra_specs, cout, R):
    grid = (R // _BLK,)
    in_specs = [
        pl.BlockSpec((_BLK, 64), lambda i: (i, 0)),
        pl.BlockSpec((_BLK, 8), lambda i: (i, 0)),
    ] + n_extra_specs
    return pl.pallas_call(
        body,
        grid=grid,
        in_specs=in_specs,
        out_specs=_full((8, cout)),
        out_shape=jax.ShapeDtypeStruct((8, cout), jnp.float32),
        scratch_shapes=[pltpu.VMEM((8, cout), jnp.float32)],
    )


def _fold(stats, g, be, R):
    s1 = stats[0]
    s2 = stats[1]
    mean = s1 / R
    var = s2 / R - mean * mean
    scale = g / jnp.sqrt(var + _EPS)
    shift = be - mean * scale
    return scale[None, :], shift[None, :]


def kernel(xyz, points, new_xyz, W0, b0, g0, be0, W1, b1, g1, be1,
           W2, b2, g2, be2):
    B, N, Cin = points.shape
    S = new_xyz.shape[1]
    R = B * S * _K

    xyz_t = jnp.transpose(xyz, (0, 2, 1)).reshape(B, 3 * N)
    nxyz_t = jnp.transpose(new_xyz, (0, 2, 1)).reshape(B, 3 * S)
    pts_flat = points.reshape(B * N, Cin)

    d2_flat = _d2_call(new_xyz, xyz).reshape(B * S, N)
    feat, rel_flat = _sc_ball_gather(d2_flat, xyz_t, nxyz_t, pts_flat,
                                     B, N, S, Cin)
    rel = rel_flat.reshape(R, 8)

    w0r = jnp.pad(W0[:, :3], ((0, 0), (0, 5))).T
    w0f = W0[:, 3:].T
    w1 = W1.T
    w2 = W2.T
    b0r = b0[None, :]
    b1r = b1[None, :]
    b2r = b2[None, :]

    wspec = [_full((64, 64)), _full((8, 64)), _full((1, 64))]
    stats1 = _stats_call(_pass_a_body, wspec, 64, R)(feat, rel, w0f, w0r, b0r)
    sc1, sh1 = _fold(stats1, g0, be0, R)

    spec_b = wspec + [_full((1, 64)), _full((1, 64)), _full((64, 128)),
                      _full((1, 128))]
    stats2 = _stats_call(_pass_b_body, spec_b, 128, R)(
        feat, rel, w0f, w0r, b0r, sc1, sh1, w1, b1r)
    sc2, sh2 = _fold(stats2, g1, be1, R)

    spec_c = spec_b + [_full((1, 128)), _full((1, 128)), _full((128, 128)),
                       _full((1, 128))]
    stats3 = _stats_call(_pass_c_body, spec_c, 128, R)(
        feat, rel, w0f, w0r, b0r, sc1, sh1, w1, b1r, sc2, sh2, w2, b2r)
    sc3, sh3 = _fold(stats3, g2, be2, R)

    spec_d = spec_c + [_full((1, 128)), _full((1, 128))]
    out = pl.pallas_call(
        _pass_d_body,
        grid=(R // _BLK,),
        in_specs=[
            pl.BlockSpec((_BLK, 64), lambda i: (i, 0)),
            pl.BlockSpec((_BLK, 8), lambda i: (i, 0)),
        ] + spec_d,
        out_specs=pl.BlockSpec((_BLK // _K, 128), lambda i: (i, 0)),
        out_shape=jax.ShapeDtypeStruct((B * S, 128), jnp.float32),
    )(feat, rel, w0f, w0r, b0r, sc1, sh1, w1, b1r, sc2, sh2, w2, b2r,
      sc3, sh3)

    new_points = jnp.transpose(out.reshape(B, S, 128), (0, 2, 1))
    return new_xyz, new_points


# speedup vs baseline: 8.8314x; 1.0792x over previous
import functools

import jax
import jax.numpy as jnp
from jax import lax
from jax.experimental import pallas as pl
from jax.experimental.pallas import tpu as pltpu
from jax.experimental.pallas import tpu_sc as plsc

_R2 = 0.2 * 0.2
_K = 32
_EPS = 1e-5
_L = 16


def _d2_body(q_ref, p_ref, o_ref):
    q = q_ref[0]
    p = p_ref[0]
    dt = jax.lax.dot_general(q, p, (((1,), (1,)), ((), ())),
                             preferred_element_type=jnp.float32)
    qq = q * q
    pp = p * p
    qn = (qq[:, 0:1] + qq[:, 1:2]) + qq[:, 2:3]
    pn = (pp[:, 0:1] + pp[:, 1:2]) + pp[:, 2:3]
    o_ref[0] = (qn + pn.T) - 2.0 * dt


def _d2_call(new_xyz, xyz):
    B, N, _ = xyz.shape
    S = new_xyz.shape[1]
    return pl.pallas_call(
        _d2_body,
        grid=(B,),
        in_specs=[pl.BlockSpec((1, S, 3), lambda i: (i, 0, 0)),
                  pl.BlockSpec((1, N, 3), lambda i: (i, 0, 0))],
        out_specs=pl.BlockSpec((1, S, N), lambda i: (i, 0, 0)),
        out_shape=jax.ShapeDtypeStruct((B, S, N), jnp.float32),
    )(new_xyz, xyz)


def _sc_ball_gather(d2_flat, xyz_t, nxyz_t, pts_flat, B, N, S, Cin):
    QW = (B * S) // 32
    NCHUNK = N // _L
    R = B * S * _K

    mesh = plsc.VectorSubcoreMesh(core_axis_name="c", subcore_axis_name="s")

    def body(d2_hbm, xyz_hbm, nxyz_hbm, pts_hbm, feat_out, rel_out,
             xyz_v, nx_v, buf_v, idx_v, rel_v, feat_v, idxg_v, d2_v,
             sem, sem0, sem1):
        wid = lax.axis_index("s") * 2 + lax.axis_index("c")
        b = wid // 2
        half = wid % 2
        qbase = b * S + half * QW

        pltpu.sync_copy(xyz_hbm.at[b], xyz_v)
        for c in range(3):
            pltpu.sync_copy(nxyz_hbm.at[b, pl.ds(c * S + half * QW, QW)],
                            nx_v.at[pl.ds(c * QW, QW)])

        zf = jnp.zeros((_L,), jnp.float32)
        for i in range(_K * 8 // _L):
            rel_v[pl.ds(i * _L, _L)] = zf

        lane = lax.iota(jnp.int32, _L)
        zero16 = jnp.zeros((_L,), jnp.int32)

        def per_query(q, dbase):
            qsp = lax.broadcast(q, (_L,))
            qx = plsc.load_gather(nx_v, [qsp])
            qy = plsc.load_gather(nx_v, [qsp + QW])
            qz = plsc.load_gather(nx_v, [qsp + 2 * QW])

            def cond(st):
                i, cur = st
                return jnp.logical_and(cur < _K, i < NCHUNK)

            def scan_chunk(st):
                i, cur = st
                d2 = d2_v[pl.ds(dbase + i * _L, _L)]
                m = d2 <= _R2
                iv = lane + i * _L
                plsc.store_compressed(buf_v.at[pl.ds(cur, _L)], iv, mask=m)
                cnt = jnp.sum(m.astype(jnp.int32))
                return i + 1, cur + cnt

            _, cursor = lax.while_loop(cond, scan_chunk, (0, 0))

            cvec = lax.broadcast(cursor, (_L,))
            far = zero16 + (N - 1)

            for h in range(_K // _L):
                jvec = lane + h * _L
                jsel = jnp.where(jvec < cvec, jvec, 0)
                gihalf = plsc.load_gather(buf_v, [jsel])
                gi = jnp.where(cvec > 0, gihalf, far)
                gx = plsc.load_gather(xyz_v, [gi])
                gy = plsc.load_gather(xyz_v, [gi + N])
                gz = plsc.load_gather(xyz_v, [gi + 2 * N])
                pos = (jvec * 8)
                plsc.store_scatter(rel_v, [pos], gx - qx)
                plsc.store_scatter(rel_v, [pos + 1], gy - qy)
                plsc.store_scatter(rel_v, [pos + 2], gz - qz)
                idx_v[pl.ds(q * _K + h * _L, _L)] = gi + b * N

            pltpu.sync_copy(rel_v, rel_out.at[pl.ds((qbase + q) * _K * 8, _K * 8)])

        pltpu.async_copy(d2_hbm.at[qbase], d2_v.at[pl.ds(0, N)], sem0)

        def pair(g, _):
            q0 = 2 * g
            pltpu.make_async_copy(d2_hbm.at[qbase + q0],
                                  d2_v.at[pl.ds(0, N)], sem0).wait()
            pltpu.async_copy(d2_hbm.at[qbase + q0 + 1],
                             d2_v.at[pl.ds(N, N)], sem1)
            per_query(q0, 0)
            q1 = q0 + 1
            pltpu.make_async_copy(d2_hbm.at[qbase + q1],
                                  d2_v.at[pl.ds(N, N)], sem1).wait()

            @pl.when(q1 + 1 < QW)
            def _prefetch():
                pltpu.async_copy(d2_hbm.at[qbase + q1 + 1],
                                 d2_v.at[pl.ds(0, N)], sem0)

            per_query(q1, N)
            return _

        lax.fori_loop(0, QW // 2, pair, 0)

        GR = 128
        rowbase = qbase * _K

        def per_gather(g, _):
            for t in range(GR // _L):
                idxg_v[pl.ds(t * _L, _L)] = idx_v[pl.ds(g * GR + t * _L, _L)]
            pltpu.async_copy(pts_hbm.at[idxg_v], feat_v, sem).wait()
            pltpu.sync_copy(feat_v, feat_out.at[pl.ds(rowbase + g * GR, GR)])
            return _

        lax.fori_loop(0, (QW * _K) // GR, per_gather, 0)

    call = pl.kernel(
        body,
        out_type=(
            jax.ShapeDtypeStruct((R, Cin), jnp.float32),
            jax.ShapeDtypeStruct((R * 8,), jnp.float32),
        ),
        mesh=mesh,
        compiler_params=pltpu.CompilerParams(needs_layout_passes=False,
                                             use_tc_tiling_on_sc=False),
        scratch_types=[
            pltpu.VMEM((3 * N,), jnp.float32),
            pltpu.VMEM((3 * QW,), jnp.float32),
            pltpu.VMEM((64,), jnp.int32),
            pltpu.VMEM((QW * _K,), jnp.int32),
            pltpu.VMEM((_K * 8,), jnp.float32),
            pltpu.VMEM((128, Cin), jnp.float32),
            pltpu.VMEM((128,), jnp.int32),
            pltpu.VMEM((2 * N,), jnp.float32),
            pltpu.SemaphoreType.DMA,
            pltpu.SemaphoreType.DMA,
            pltpu.SemaphoreType.DMA,
        ],
    )
    return call(d2_flat, xyz_t, nxyz_t, pts_flat)


_BLK = 8192


def _h_first(feat, rel, w0f, w0r, b0):
    h = jnp.dot(feat, w0f, preferred_element_type=jnp.float32)
    h = h + jnp.dot(rel, w0r, preferred_element_type=jnp.float32)
    return h + b0


def _stats_tail(i, h, out_ref, acc_ref):
    @pl.when(i == 0)
    def _():
        acc_ref[...] = jnp.zeros_like(acc_ref)

    acc_ref[0:1] += jnp.sum(h, axis=0, keepdims=True)
    acc_ref[1:2] += jnp.sum(h * h, axis=0, keepdims=True)

    @pl.when(i == pl.num_programs(0) - 1)
    def _():
        out_ref[...] = acc_ref[...]


def _pass_a_body(feat_ref, rel_ref, w0f, w0r, b0, out_ref, acc_ref):
    i = pl.program_id(0)
    h = _h_first(feat_ref[...], rel_ref[...], w0f[...], w0r[...], b0[...])
    _stats_tail(i, h, out_ref, acc_ref)


def _pass_b_body(feat_ref, rel_ref, w0f, w0r, b0, sc1, sh1, w1, b1,
                 out_ref, acc_ref):
    i = pl.program_id(0)
    h = _h_first(feat_ref[...], rel_ref[...], w0f[...], w0r[...], b0[...])
    y1 = jnp.maximum(h * sc1[...] + sh1[...], 0.0)
    h2 = jnp.dot(y1, w1[...], preferred_element_type=jnp.float32) + b1[...]
    _stats_tail(i, h2, out_ref, acc_ref)


def _pass_c_body(feat_ref, rel_ref, w0f, w0r, b0, sc1, sh1, w1, b1, sc2, sh2,
                 w2, b2, out_ref, acc_ref):
    i = pl.program_id(0)
    h = _h_first(feat_ref[...], rel_ref[...], w0f[...], w0r[...], b0[...])
    y1 = jnp.maximum(h * sc1[...] + sh1[...], 0.0)
    h2 = jnp.dot(y1, w1[...], preferred_element_type=jnp.float32) + b1[...]
    y2 = jnp.maximum(h2 * sc2[...] + sh2[...], 0.0)
    h3 = jnp.dot(y2, w2[...], preferred_element_type=jnp.float32) + b2[...]
    _stats_tail(i, h3, out_ref, acc_ref)


def _pass_d_body(feat_ref, rel_ref, w0f, w0r, b0, sc1, sh1, w1, b1, sc2, sh2,
                 w2, b2, sc3, sh3, out_ref):
    h = _h_first(feat_ref[...], rel_ref[...], w0f[...], w0r[...], b0[...])
    y1 = jnp.maximum(h * sc1[...] + sh1[...], 0.0)
    h2 = jnp.dot(y1, w1[...], preferred_element_type=jnp.float32) + b1[...]
    y2 = jnp.maximum(h2 * sc2[...] + sh2[...], 0.0)
    h3 = jnp.dot(y2, w2[...], preferred_element_type=jnp.float32) + b2[...]
    y3 = jnp.maximum(h3 * sc3[...] + sh3[...], 0.0)
    co = y3.shape[-1]
    out_ref[...] = jnp.max(y3.reshape(_BLK // _K, _K, co), axis=1)


def _full(shape):
    return pl.BlockSpec(shape, lambda i: (0,) * len(shape))


def _stats_call(body, n_extra_specs, cout, R):
    grid = (R // _BLK,)
    in_specs = [
        pl.BlockSpec((_BLK, 64), lambda i: (i, 0)),
        pl.BlockSpec((_BLK, 8), lambda i: (i, 0)),
    ] + n_extra_specs
    return pl.pallas_call(
        body,
        grid=grid,
        in_specs=in_specs,
        out_specs=_full((8, cout)),
        out_shape=jax.ShapeDtypeStruct((8, cout), jnp.float32),
        scratch_shapes=[pltpu.VMEM((8, cout), jnp.float32)],
    )


def _fold(stats, g, be, R):
    s1 = stats[0]
    s2 = stats[1]
    mean = s1 / R
    var = s2 / R - mean * mean
    scale = g / jnp.sqrt(var + _EPS)
    shift = be - mean * scale
    return scale[None, :], shift[None, :]


def kernel(xyz, points, new_xyz, W0, b0, g0, be0, W1, b1, g1, be1,
           W2, b2, g2, be2):
    B, N, Cin = points.shape
    S = new_xyz.shape[1]
    R = B * S * _K

    xyz_t = jnp.transpose(xyz, (0, 2, 1)).reshape(B, 3 * N)
    nxyz_t = jnp.transpose(new_xyz, (0, 2, 1)).reshape(B, 3 * S)
    pts_flat = points.reshape(B * N, Cin)

    d2_flat = _d2_call(new_xyz, xyz).reshape(B * S, N)
    feat, rel_flat = _sc_ball_gather(d2_flat, xyz_t, nxyz_t, pts_flat,
                                     B, N, S, Cin)
    rel = rel_flat.reshape(R, 8)

    w0r = jnp.pad(W0[:, :3], ((0, 0), (0, 5))).T
    w0f = W0[:, 3:].T
    w1 = W1.T
    w2 = W2.T
    b0r = b0[None, :]
    b1r = b1[None, :]
    b2r = b2[None, :]

    wspec = [_full((64, 64)), _full((8, 64)), _full((1, 64))]
    stats1 = _stats_call(_pass_a_body, wspec, 64, R)(feat, rel, w0f, w0r, b0r)
    sc1, sh1 = _fold(stats1, g0, be0, R)

    spec_b = wspec + [_full((1, 64)), _full((1, 64)), _full((64, 128)),
                      _full((1, 128))]
    stats2 = _stats_call(_pass_b_body, spec_b, 128, R)(
        feat, rel, w0f, w0r, b0r, sc1, sh1, w1, b1r)
    sc2, sh2 = _fold(stats2, g1, be1, R)

    spec_c = spec_b + [_full((1, 128)), _full((1, 128)), _full((128, 128)),
                       _full((1, 128))]
    stats3 = _stats_call(_pass_c_body, spec_c, 128, R)(
        feat, rel, w0f, w0r, b0r, sc1, sh1, w1, b1r, sc2, sh2, w2, b2r)
    sc3, sh3 = _fold(stats3, g2, be2, R)

    spec_d = spec_c + [_full((1, 128)), _full((1, 128))]
    out = pl.pallas_call(
        _pass_d_body,
        grid=(R // _BLK,),
        in_specs=[
            pl.BlockSpec((_BLK, 64), lambda i: (i, 0)),
            pl.BlockSpec((_BLK, 8), lambda i: (i, 0)),
        ] + spec_d,
        out_specs=pl.BlockSpec((_BLK // _K, 128), lambda i: (i, 0)),
        out_shape=jax.ShapeDtypeStruct((B * S, 128), jnp.float32),
    )(feat, rel, w0f, w0r, b0r, sc1, sh1, w1, b1r, sc2, sh2, w2, b2r,
      sc3, sh3)

    new_points = jnp.transpose(out.reshape(B, S, 128), (0, 2, 1))
    return new_xyz, new_points


# speedup vs baseline: 11.6061x; 1.3142x over previous
import functools

import jax
import jax.numpy as jnp
from jax import lax
from jax.experimental import pallas as pl
from jax.experimental.pallas import tpu as pltpu
from jax.experimental.pallas import tpu_sc as plsc

_R2 = 0.2 * 0.2
_K = 32
_EPS = 1e-5
_L = 16


def _d2_body(q_ref, p_ref, o_ref):
    q = q_ref[0]
    p = p_ref[0]
    dt = jax.lax.dot_general(q, p, (((1,), (1,)), ((), ())),
                             preferred_element_type=jnp.float32)
    qq = q * q
    pp = p * p
    qn = (qq[:, 0:1] + qq[:, 1:2]) + qq[:, 2:3]
    pn = (pp[:, 0:1] + pp[:, 1:2]) + pp[:, 2:3]
    o_ref[0] = (qn + pn.T) - 2.0 * dt


def _d2_call(new_xyz, xyz):
    B, N, _ = xyz.shape
    S = new_xyz.shape[1]
    return pl.pallas_call(
        _d2_body,
        grid=(B,),
        in_specs=[pl.BlockSpec((1, S, 3), lambda i: (i, 0, 0)),
                  pl.BlockSpec((1, N, 3), lambda i: (i, 0, 0))],
        out_specs=pl.BlockSpec((1, S, N), lambda i: (i, 0, 0)),
        out_shape=jax.ShapeDtypeStruct((B, S, N), jnp.float32),
    )(new_xyz, xyz)


def _sc_ball_gather(d2_flat, xyz_t, nxyz_t, pts_flat, B, N, S, Cin):
    QW = (B * S) // 32
    NCHUNK = N // _L
    R = B * S * _K

    mesh = plsc.VectorSubcoreMesh(core_axis_name="c", subcore_axis_name="s")

    def body(d2_hbm, xyz_hbm, nxyz_hbm, pts_hbm, feat_out, rel_out,
             xyz_v, nx_v, buf_v, idx_v, rel_v, feat_v, idxg_v, d2_v,
             sem, sem0, sem1):
        wid = lax.axis_index("s") * 2 + lax.axis_index("c")
        b = wid // 2
        half = wid % 2
        qbase = b * S + half * QW

        pltpu.sync_copy(xyz_hbm.at[b], xyz_v)
        for c in range(3):
            pltpu.sync_copy(nxyz_hbm.at[b, pl.ds(c * S + half * QW, QW)],
                            nx_v.at[pl.ds(c * QW, QW)])

        zf = jnp.zeros((_L,), jnp.float32)
        for i in range(_K * 8 // _L):
            rel_v[pl.ds(i * _L, _L)] = zf

        lane = lax.iota(jnp.int32, _L)
        zero16 = jnp.zeros((_L,), jnp.int32)

        def per_query(q, dbase):
            qsp = lax.broadcast(q, (_L,))
            qx = plsc.load_gather(nx_v, [qsp])
            qy = plsc.load_gather(nx_v, [qsp + QW])
            qz = plsc.load_gather(nx_v, [qsp + 2 * QW])

            U = 8

            def cond(st):
                i, cur = st
                return jnp.logical_and(cur < _K, i < NCHUNK)

            def scan_group(st):
                i, cur = st
                ms, cs = [], []
                for u in range(U):
                    d2 = d2_v[pl.ds(dbase + (i + u) * _L, _L)]
                    m = d2 <= _R2
                    ms.append(m)
                    cs.append(jnp.sum(m.astype(jnp.int32)))
                o = cur
                for u in range(U):
                    iv = lane + (i + u) * _L
                    plsc.store_compressed(buf_v.at[pl.ds(o, _L)], iv,
                                          mask=ms[u])
                    o = o + cs[u]
                return i + U, o

            _, cursor = lax.while_loop(cond, scan_group, (0, 0))

            cvec = lax.broadcast(cursor, (_L,))
            far = zero16 + (N - 1)

            for h in range(_K // _L):
                jvec = lane + h * _L
                jsel = jnp.where(jvec < cvec, jvec, 0)
                gihalf = plsc.load_gather(buf_v, [jsel])
                gi = jnp.where(cvec > 0, gihalf, far)
                gx = plsc.load_gather(xyz_v, [gi])
                gy = plsc.load_gather(xyz_v, [gi + N])
                gz = plsc.load_gather(xyz_v, [gi + 2 * N])
                pos = (jvec * 8)
                plsc.store_scatter(rel_v, [pos], gx - qx)
                plsc.store_scatter(rel_v, [pos + 1], gy - qy)
                plsc.store_scatter(rel_v, [pos + 2], gz - qz)
                idx_v[pl.ds(q * _K + h * _L, _L)] = gi + b * N

            pltpu.sync_copy(rel_v, rel_out.at[pl.ds((qbase + q) * _K * 8, _K * 8)])

        pltpu.async_copy(d2_hbm.at[qbase], d2_v.at[pl.ds(0, N)], sem0)

        def pair(g, _):
            q0 = 2 * g
            pltpu.make_async_copy(d2_hbm.at[qbase + q0],
                                  d2_v.at[pl.ds(0, N)], sem0).wait()
            pltpu.async_copy(d2_hbm.at[qbase + q0 + 1],
                             d2_v.at[pl.ds(N, N)], sem1)
            per_query(q0, 0)
            q1 = q0 + 1
            pltpu.make_async_copy(d2_hbm.at[qbase + q1],
                                  d2_v.at[pl.ds(N, N)], sem1).wait()

            @pl.when(q1 + 1 < QW)
            def _prefetch():
                pltpu.async_copy(d2_hbm.at[qbase + q1 + 1],
                                 d2_v.at[pl.ds(0, N)], sem0)

            per_query(q1, N)
            return _

        lax.fori_loop(0, QW // 2, pair, 0)

        GR = 128
        rowbase = qbase * _K

        def per_gather(g, _):
            for t in range(GR // _L):
                idxg_v[pl.ds(t * _L, _L)] = idx_v[pl.ds(g * GR + t * _L, _L)]
            pltpu.async_copy(pts_hbm.at[idxg_v], feat_v, sem).wait()
            pltpu.sync_copy(feat_v, feat_out.at[pl.ds(rowbase + g * GR, GR)])
            return _

        lax.fori_loop(0, (QW * _K) // GR, per_gather, 0)

    call = pl.kernel(
        body,
        out_type=(
            jax.ShapeDtypeStruct((R, Cin), jnp.float32),
            jax.ShapeDtypeStruct((R * 8,), jnp.float32),
        ),
        mesh=mesh,
        compiler_params=pltpu.CompilerParams(needs_layout_passes=False,
                                             use_tc_tiling_on_sc=False),
        scratch_types=[
            pltpu.VMEM((3 * N,), jnp.float32),
            pltpu.VMEM((3 * QW,), jnp.float32),
            pltpu.VMEM((192,), jnp.int32),
            pltpu.VMEM((QW * _K,), jnp.int32),
            pltpu.VMEM((_K * 8,), jnp.float32),
            pltpu.VMEM((128, Cin), jnp.float32),
            pltpu.VMEM((128,), jnp.int32),
            pltpu.VMEM((2 * N,), jnp.float32),
            pltpu.SemaphoreType.DMA,
            pltpu.SemaphoreType.DMA,
            pltpu.SemaphoreType.DMA,
        ],
    )
    return call(d2_flat, xyz_t, nxyz_t, pts_flat)


_BLK = 8192


def _h_first(feat, rel, w0f, w0r, b0):
    h = jnp.dot(feat, w0f, preferred_element_type=jnp.float32)
    h = h + jnp.dot(rel, w0r, preferred_element_type=jnp.float32)
    return h + b0


def _stats_tail(i, h, out_ref, acc_ref):
    @pl.when(i == 0)
    def _():
        acc_ref[...] = jnp.zeros_like(acc_ref)

    acc_ref[0:1] += jnp.sum(h, axis=0, keepdims=True)
    acc_ref[1:2] += jnp.sum(h * h, axis=0, keepdims=True)

    @pl.when(i == pl.num_programs(0) - 1)
    def _():
        out_ref[...] = acc_ref[...]


def _pass_a_body(feat_ref, rel_ref, w0f, w0r, b0, out_ref, acc_ref):
    i = pl.program_id(0)
    h = _h_first(feat_ref[...], rel_ref[...], w0f[...], w0r[...], b0[...])
    _stats_tail(i, h, out_ref, acc_ref)


def _pass_b_body(feat_ref, rel_ref, w0f, w0r, b0, sc1, sh1, w1, b1,
                 out_ref, acc_ref):
    i = pl.program_id(0)
    h = _h_first(feat_ref[...], rel_ref[...], w0f[...], w0r[...], b0[...])
    y1 = jnp.maximum(h * sc1[...] + sh1[...], 0.0)
    h2 = jnp.dot(y1, w1[...], preferred_element_type=jnp.float32) + b1[...]
    _stats_tail(i, h2, out_ref, acc_ref)


def _pass_c_body(feat_ref, rel_ref, w0f, w0r, b0, sc1, sh1, w1, b1, sc2, sh2,
                 w2, b2, out_ref, acc_ref):
    i = pl.program_id(0)
    h = _h_first(feat_ref[...], rel_ref[...], w0f[...], w0r[...], b0[...])
    y1 = jnp.maximum(h * sc1[...] + sh1[...], 0.0)
    h2 = jnp.dot(y1, w1[...], preferred_element_type=jnp.float32) + b1[...]
    y2 = jnp.maximum(h2 * sc2[...] + sh2[...], 0.0)
    h3 = jnp.dot(y2, w2[...], preferred_element_type=jnp.float32) + b2[...]
    _stats_tail(i, h3, out_ref, acc_ref)


def _pass_d_body(feat_ref, rel_ref, w0f, w0r, b0, sc1, sh1, w1, b1, sc2, sh2,
                 w2, b2, sc3, sh3, out_ref):
    h = _h_first(feat_ref[...], rel_ref[...], w0f[...], w0r[...], b0[...])
    y1 = jnp.maximum(h * sc1[...] + sh1[...], 0.0)
    h2 = jnp.dot(y1, w1[...], preferred_element_type=jnp.float32) + b1[...]
    y2 = jnp.maximum(h2 * sc2[...] + sh2[...], 0.0)
    h3 = jnp.dot(y2, w2[...], preferred_element_type=jnp.float32) + b2[...]
    y3 = jnp.maximum(h3 * sc3[...] + sh3[...], 0.0)
    co = y3.shape[-1]
    out_ref[...] = jnp.max(y3.reshape(_BLK // _K, _K, co), axis=1)


def _full(shape):
    return pl.BlockSpec(shape, lambda i: (0,) * len(shape))


def _stats_call(body, n_extra_specs, cout, R):
    grid = (R // _BLK,)
    in_specs = [
        pl.BlockSpec((_BLK, 64), lambda i: (i, 0)),
        pl.BlockSpec((_BLK, 8), lambda i: (i, 0)),
    ] + n_extra_specs
    return pl.pallas_call(
        body,
        grid=grid,
        in_specs=in_specs,
        out_specs=_full((8, cout)),
        out_shape=jax.ShapeDtypeStruct((8, cout), jnp.float32),
        scratch_shapes=[pltpu.VMEM((8, cout), jnp.float32)],
    )


def _fold(stats, g, be, R):
    s1 = stats[0]
    s2 = stats[1]
    mean = s1 / R
    var = s2 / R - mean * mean
    scale = g / jnp.sqrt(var + _EPS)
    shift = be - mean * scale
    return scale[None, :], shift[None, :]


def kernel(xyz, points, new_xyz, W0, b0, g0, be0, W1, b1, g1, be1,
           W2, b2, g2, be2):
    B, N, Cin = points.shape
    S = new_xyz.shape[1]
    R = B * S * _K

    xyz_t = jnp.transpose(xyz, (0, 2, 1)).reshape(B, 3 * N)
    nxyz_t = jnp.transpose(new_xyz, (0, 2, 1)).reshape(B, 3 * S)
    pts_flat = points.reshape(B * N, Cin)

    d2_flat = _d2_call(new_xyz, xyz).reshape(B * S, N)
    feat, rel_flat = _sc_ball_gather(d2_flat, xyz_t, nxyz_t, pts_flat,
                                     B, N, S, Cin)
    rel = rel_flat.reshape(R, 8)

    w0r = jnp.pad(W0[:, :3], ((0, 0), (0, 5))).T
    w0f = W0[:, 3:].T
    w1 = W1.T
    w2 = W2.T
    b0r = b0[None, :]
    b1r = b1[None, :]
    b2r = b2[None, :]

    wspec = [_full((64, 64)), _full((8, 64)), _full((1, 64))]
    stats1 = _stats_call(_pass_a_body, wspec, 64, R)(feat, rel, w0f, w0r, b0r)
    sc1, sh1 = _fold(stats1, g0, be0, R)

    spec_b = wspec + [_full((1, 64)), _full((1, 64)), _full((64, 128)),
                      _full((1, 128))]
    stats2 = _stats_call(_pass_b_body, spec_b, 128, R)(
        feat, rel, w0f, w0r, b0r, sc1, sh1, w1, b1r)
    sc2, sh2 = _fold(stats2, g1, be1, R)

    spec_c = spec_b + [_full((1, 128)), _full((1, 128)), _full((128, 128)),
                       _full((1, 128))]
    stats3 = _stats_call(_pass_c_body, spec_c, 128, R)(
        feat, rel, w0f, w0r, b0r, sc1, sh1, w1, b1r, sc2, sh2, w2, b2r)
    sc3, sh3 = _fold(stats3, g2, be2, R)

    spec_d = spec_c + [_full((1, 128)), _full((1, 128))]
    out = pl.pallas_call(
        _pass_d_body,
        grid=(R // _BLK,),
        in_specs=[
            pl.BlockSpec((_BLK, 64), lambda i: (i, 0)),
            pl.BlockSpec((_BLK, 8), lambda i: (i, 0)),
        ] + spec_d,
        out_specs=pl.BlockSpec((_BLK // _K, 128), lambda i: (i, 0)),
        out_shape=jax.ShapeDtypeStruct((B * S, 128), jnp.float32),
    )(feat, rel, w0f, w0r, b0r, sc1, sh1, w1, b1r, sc2, sh2, w2, b2r,
      sc3, sh3)

    new_points = jnp.transpose(out.reshape(B, S, 128), (0, 2, 1))
    return new_xyz, new_points
